# baseline (speedup 1.0000x reference)
import functools

import jax
import jax.numpy as jnp
from jax import lax
from jax.experimental import pallas as pl
from jax.experimental.pallas import tpu as pltpu
from jax.experimental.pallas import tpu_sc as plsc

D = 32
K = 10
LANES = 16
NC, NS = 2, 16
NW = NC * NS
OUT_PAD = 16


def _sc_body(users_hbm, pos_hbm, negt_hbm, utab_hbm, itab_hbm, out_hbm,
             uidx_v, pidx_v, nidx_v, urows_v, prows_v, nrows_v, out_v, sem,
             *, bpw):
    wid = lax.axis_index("s") * NC + lax.axis_index("c")
    base = wid * bpw

    pltpu.sync_copy(users_hbm.at[pl.ds(base, bpw)], uidx_v)
    pltpu.sync_copy(pos_hbm.at[pl.ds(base, bpw)], pidx_v)
    pltpu.sync_copy(negt_hbm.at[:, pl.ds(base, bpw)], nidx_v)

    cps = [pltpu.async_copy(utab_hbm.at[uidx_v], urows_v, sem),
           pltpu.async_copy(itab_hbm.at[pidx_v], prows_v, sem)]
    for k in range(K):
        cps.append(pltpu.async_copy(itab_hbm.at[nidx_v.at[k]],
                                    nrows_v.at[k], sem))
    for cp in cps:
        cp.wait()

    for g in range(bpw // LANES):
        b_idx = lax.iota(jnp.int32, LANES) + g * LANES
        kcol = [jnp.full((LANES,), k, jnp.int32) for k in range(K)]

        def dim_step(d, accs):
            dspl = jnp.full((LANES,), d, jnp.int32)
            u = plsc.load_gather(urows_v, [b_idx, dspl])
            p = plsc.load_gather(prows_v, [b_idx, dspl])
            new = [accs[0] + u * p]
            for k in range(K):
                n = plsc.load_gather(nrows_v, [kcol[k], b_idx, dspl])
                new.append(accs[k + 1] + u * n)
            return tuple(new)

        zeros = tuple(jnp.zeros((LANES,), jnp.float32) for _ in range(K + 1))
        accs = lax.fori_loop(0, D, dim_step, zeros)
        for k in range(K + 1):
            plsc.store_scatter(out_v, [b_idx, jnp.full((LANES,), k, jnp.int32)],
                               accs[k])

    pltpu.sync_copy(out_v, out_hbm.at[pl.ds(base, bpw)])


@functools.partial(jax.jit, static_argnums=())
def kernel(users, pos_items, neg_items, user_table, item_table):
    batch = users.shape[0]
    bpw = batch // NW
    neg_t = neg_items.T

    mesh = plsc.VectorSubcoreMesh(core_axis_name="c", subcore_axis_name="s")
    run = functools.partial(
        pl.kernel,
        mesh=mesh,
        compiler_params=pltpu.CompilerParams(
            needs_layout_passes=False, use_tc_tiling_on_sc=False),
        out_type=jax.ShapeDtypeStruct((batch, OUT_PAD), jnp.float32),
        scratch_types=[
            pltpu.VMEM((bpw,), jnp.int32),
            pltpu.VMEM((bpw,), jnp.int32),
            pltpu.VMEM((K, bpw), jnp.int32),
            pltpu.VMEM((bpw, D), jnp.float32),
            pltpu.VMEM((bpw, D), jnp.float32),
            pltpu.VMEM((K, bpw, D), jnp.float32),
            pltpu.VMEM((bpw, OUT_PAD), jnp.float32),
            pltpu.SemaphoreType.DMA,
        ],
    )(functools.partial(_sc_body, bpw=bpw))
    padded = run(users, pos_items, neg_t, user_table, item_table)
    return padded[:, :K + 1]
